# baseline (speedup 1.0000x reference)
import jax
import jax.numpy as jnp
from jax import lax
from jax.experimental import pallas as pl
from jax.experimental.pallas import tpu as pltpu
from jax.experimental.pallas import tpu_sc as plsc

B, D, V = 16384, 128, 1000000

NC = 2
NS = 16
NL = 16
NW = NC * NS
BPW = B // NW
CH = 128
K = BPW // CH
NBUF = 2
NACC = 1
_INV = 1.0 / float(B * D)


def _sc_body(idx_hbm, table_hbm, l_hbm, r_hbm, out_hbm,
             idx_v, alpha_v, lbuf, rbuf, acc_v,
             sem_a0, sem_a1, sem_a2, sem_a3, sem_d0, sem_d1):
    wid = lax.axis_index("s") * NC + lax.axis_index("c")
    base = wid * BPW
    sems = (sem_d0, sem_d1)
    asems = (sem_a0, sem_a1, sem_a2, sem_a3)

    def fire_alpha(j):
        return pltpu.async_copy(table_hbm.at[idx_v.at[j]], alpha_v.at[j],
                                asems[j])

    d_cps = {}

    def fire(c):
        s = c % NBUF
        d_cps[c] = (
            pltpu.async_copy(l_hbm.at[pl.ds(base + c * CH, CH), :], lbuf.at[s],
                             sems[s]),
            pltpu.async_copy(r_hbm.at[pl.ds(base + c * CH, CH), :], rbuf.at[s],
                             sems[s]),
        )

    fire(0)
    pltpu.sync_copy(idx_hbm.at[pl.ds(wid * K, K)], idx_v)
    a_cps = {j: fire_alpha(j) for j in range(K)}
    fire(1)

    accs = tuple(jnp.zeros((NL,), jnp.float32) for _ in range(NACC))
    for c in range(K):
        a_cps[c].wait()
        for cp in d_cps[c]:
            cp.wait()
        s = c % NBUF

        def group_body(g, at, s=s, c=c):
            a16 = alpha_v[c, pl.ds(g * NL, NL)]
            al = list(at)
            for j in range(NL):
                a_s = a16[j]
                r = g * NL + j
                for gg in range(D // NL):
                    lv = lbuf[s, r, pl.ds(gg * NL, NL)]
                    rv = rbuf[s, r, pl.ds(gg * NL, NL)]
                    al[gg % NACC] = al[gg % NACC] + jnp.abs(lv - a_s * rv)
            return tuple(al)

        accs = lax.fori_loop(0, CH // NL, group_body, accs, unroll=1)
        if c + NBUF < K:
            fire(c + NBUF)

    acc_v[...] = accs[0]
    pltpu.sync_copy(acc_v, out_hbm.at[wid])


_sc_loss = pl.kernel(
    _sc_body,
    mesh=plsc.VectorSubcoreMesh(core_axis_name="c", subcore_axis_name="s"),
    out_type=jax.ShapeDtypeStruct((NW, NL), jnp.float32),
    scratch_types=[
        pltpu.VMEM((K, CH), jnp.int32),
        pltpu.VMEM((K, CH), jnp.float32),
        pltpu.VMEM((NBUF, CH, D), jnp.float32),
        pltpu.VMEM((NBUF, CH, D), jnp.float32),
        pltpu.VMEM((NL,), jnp.float32),
        pltpu.SemaphoreType.DMA,
        pltpu.SemaphoreType.DMA,
        pltpu.SemaphoreType.DMA,
        pltpu.SemaphoreType.DMA,
        pltpu.SemaphoreType.DMA,
        pltpu.SemaphoreType.DMA,
    ],
)


def _fin_body(p_ref, out_ref):
    out_ref[0, 0] = jnp.sum(p_ref[...]) * _INV


_finish = pl.pallas_call(
    _fin_body,
    out_specs=pl.BlockSpec(memory_space=pltpu.SMEM),
    out_shape=jax.ShapeDtypeStruct((1, 1), jnp.float32),
)


def kernel(emd_l, emd_r, player, alpha_weight):
    idx = player.astype(jnp.int32).reshape(NW * K, CH)
    table = alpha_weight.reshape(V)
    parts = _sc_loss(idx, table, emd_l, emd_r)
    return _finish(parts)[0, 0]


# speedup vs baseline: 1.0059x; 1.0059x over previous
import jax
import jax.numpy as jnp
from jax import lax
from jax.experimental import pallas as pl
from jax.experimental.pallas import tpu as pltpu
from jax.experimental.pallas import tpu_sc as plsc

B, D, V = 16384, 128, 1000000

NC = 2
NS = 16
NL = 16
NW = NC * NS
BPW = B // NW
GC = 128
KG = BPW // GC
SIZES = (32, 96, 160, 224)
STARTS = (0, 32, 128, 288)
SLOT_ROWS = 224
AWAITS = ((0,), (), (1, 2), (3,))
_INV = 1.0 / float(B * D)


def _sc_body(idx_hbm, table_hbm, l_hbm, r_hbm, out_hbm,
             idx_v, alpha_v, lbuf, rbuf, acc_v,
             sem_a0, sem_a1, sem_a2, sem_a3, sem_c0, sem_c1, sem_c2, sem_c3):
    wid = lax.axis_index("s") * NC + lax.axis_index("c")
    base = wid * BPW
    asems = (sem_a0, sem_a1, sem_a2, sem_a3)
    dsems = (sem_c0, sem_c1, sem_c2, sem_c3)

    def fire_alpha(j):
        return pltpu.async_copy(table_hbm.at[idx_v.at[j]],
                                alpha_v.at[pl.ds(j * GC, GC)], asems[j])

    d_cps = {}

    def fire(c):
        s = c % 2
        rows = SIZES[c]
        d_cps[c] = (
            pltpu.async_copy(l_hbm.at[pl.ds(base + STARTS[c], rows), :],
                             lbuf.at[s, pl.ds(0, rows), :], dsems[c]),
            pltpu.async_copy(r_hbm.at[pl.ds(base + STARTS[c], rows), :],
                             rbuf.at[s, pl.ds(0, rows), :], dsems[c]),
        )

    fire(0)
    pltpu.sync_copy(idx_hbm.at[pl.ds(wid * KG, KG)], idx_v)
    a_cps = {j: fire_alpha(j) for j in range(KG)}
    fire(1)

    acc = jnp.zeros((NL,), jnp.float32)
    for c in range(len(SIZES)):
        for j in AWAITS[c]:
            a_cps[j].wait()
        for cp in d_cps[c]:
            cp.wait()
        if c + 2 < len(SIZES):
            fire(c + 2)
        s = c % 2

        def group_body(g, a, s=s, c=c):
            a16 = alpha_v[pl.ds(STARTS[c] + g * NL, NL)]
            for j in range(NL):
                a_s = a16[j]
                r = g * NL + j
                for gg in range(D // NL):
                    lv = lbuf[s, r, pl.ds(gg * NL, NL)]
                    rv = rbuf[s, r, pl.ds(gg * NL, NL)]
                    a = a + jnp.abs(lv - a_s * rv)
            return a

        acc = lax.fori_loop(0, SIZES[c] // NL, group_body, acc, unroll=1)

    acc_v[...] = acc
    pltpu.sync_copy(acc_v, out_hbm.at[wid])


_sc_loss = pl.kernel(
    _sc_body,
    mesh=plsc.VectorSubcoreMesh(core_axis_name="c", subcore_axis_name="s"),
    out_type=jax.ShapeDtypeStruct((NW, NL), jnp.float32),
    scratch_types=[
        pltpu.VMEM((KG, GC), jnp.int32),
        pltpu.VMEM((BPW,), jnp.float32),
        pltpu.VMEM((2, SLOT_ROWS, D), jnp.float32),
        pltpu.VMEM((2, SLOT_ROWS, D), jnp.float32),
        pltpu.VMEM((NL,), jnp.float32),
        pltpu.SemaphoreType.DMA,
        pltpu.SemaphoreType.DMA,
        pltpu.SemaphoreType.DMA,
        pltpu.SemaphoreType.DMA,
        pltpu.SemaphoreType.DMA,
        pltpu.SemaphoreType.DMA,
        pltpu.SemaphoreType.DMA,
        pltpu.SemaphoreType.DMA,
    ],
)


def _fin_body(p_ref, out_ref):
    out_ref[0, 0] = jnp.sum(p_ref[...]) * _INV


_finish = pl.pallas_call(
    _fin_body,
    out_specs=pl.BlockSpec(memory_space=pltpu.SMEM),
    out_shape=jax.ShapeDtypeStruct((1, 1), jnp.float32),
)


def kernel(emd_l, emd_r, player, alpha_weight):
    idx = player.astype(jnp.int32).reshape(NW * KG, GC)
    table = alpha_weight.reshape(V)
    parts = _sc_loss(idx, table, emd_l, emd_r)
    return _finish(parts)[0, 0]


# speedup vs baseline: 1.0084x; 1.0025x over previous
import jax
import jax.numpy as jnp
from jax import lax
from jax.experimental import pallas as pl
from jax.experimental.pallas import tpu as pltpu
from jax.experimental.pallas import tpu_sc as plsc

B, D, V = 16384, 128, 1000000

NC = 2
NS = 16
NL = 16
NW = NC * NS
BPW = B // NW
SIZES = (32, 96, 160, 224)
STARTS = (0, 32, 128, 288)
SLOT_ROWS = 224
_INV = 1.0 / float(B * D)


def _sc_body(idx_hbm, table_hbm, l_hbm, r_hbm, out_hbm,
             idx_v, alpha_v, lbuf, rbuf, acc_v,
             sem_a0, sem_a1, sem_a2, sem_a3, sem_c0, sem_c1, sem_c2, sem_c3):
    wid = lax.axis_index("s") * NC + lax.axis_index("c")
    base = wid * BPW
    asems = (sem_a0, sem_a1, sem_a2, sem_a3)
    dsems = (sem_c0, sem_c1, sem_c2, sem_c3)

    def fire_alpha(j):
        sl = pl.ds(STARTS[j], SIZES[j])
        return pltpu.async_copy(table_hbm.at[idx_v.at[sl]],
                                alpha_v.at[sl], asems[j])

    d_cps = {}

    def fire(c):
        s = c % 2
        rows = SIZES[c]
        d_cps[c] = (
            pltpu.async_copy(l_hbm.at[pl.ds(base + STARTS[c], rows), :],
                             lbuf.at[s, pl.ds(0, rows), :], dsems[c]),
            pltpu.async_copy(r_hbm.at[pl.ds(base + STARTS[c], rows), :],
                             rbuf.at[s, pl.ds(0, rows), :], dsems[c]),
        )

    fire(0)
    pltpu.sync_copy(idx_hbm.at[pl.ds(wid * BPW, BPW)], idx_v)
    a_cps = {j: fire_alpha(j) for j in range(len(SIZES))}
    fire(1)

    acc = jnp.zeros((NL,), jnp.float32)
    for c in range(len(SIZES)):
        a_cps[c].wait()
        for cp in d_cps[c]:
            cp.wait()
        if c + 2 < len(SIZES):
            fire(c + 2)
        s = c % 2

        def group_body(g, a, s=s, c=c):
            a16 = alpha_v[pl.ds(STARTS[c] + g * NL, NL)]
            for j in range(NL):
                a_s = a16[j]
                r = g * NL + j
                for gg in range(D // NL):
                    lv = lbuf[s, r, pl.ds(gg * NL, NL)]
                    rv = rbuf[s, r, pl.ds(gg * NL, NL)]
                    a = a + jnp.abs(lv - a_s * rv)
            return a

        acc = lax.fori_loop(0, SIZES[c] // NL, group_body, acc, unroll=1)

    acc_v[...] = acc
    pltpu.sync_copy(acc_v, out_hbm.at[wid])


_sc_loss = pl.kernel(
    _sc_body,
    mesh=plsc.VectorSubcoreMesh(core_axis_name="c", subcore_axis_name="s"),
    out_type=jax.ShapeDtypeStruct((NW, NL), jnp.float32),
    scratch_types=[
        pltpu.VMEM((BPW,), jnp.int32),
        pltpu.VMEM((BPW,), jnp.float32),
        pltpu.VMEM((2, SLOT_ROWS, D), jnp.float32),
        pltpu.VMEM((2, SLOT_ROWS, D), jnp.float32),
        pltpu.VMEM((NL,), jnp.float32),
        pltpu.SemaphoreType.DMA,
        pltpu.SemaphoreType.DMA,
        pltpu.SemaphoreType.DMA,
        pltpu.SemaphoreType.DMA,
        pltpu.SemaphoreType.DMA,
        pltpu.SemaphoreType.DMA,
        pltpu.SemaphoreType.DMA,
        pltpu.SemaphoreType.DMA,
    ],
)


def _fin_body(p_ref, out_ref):
    out_ref[0, 0] = jnp.sum(p_ref[...]) * _INV


_finish = pl.pallas_call(
    _fin_body,
    out_specs=pl.BlockSpec(memory_space=pltpu.SMEM),
    out_shape=jax.ShapeDtypeStruct((1, 1), jnp.float32),
)


def kernel(emd_l, emd_r, player, alpha_weight):
    idx = player.astype(jnp.int32)
    table = alpha_weight.reshape(V)
    parts = _sc_loss(idx, table, emd_l, emd_r)
    return _finish(parts)[0, 0]
